# Initial kernel scaffold; baseline (speedup 1.0000x reference)
import jax
import jax.numpy as jnp
from jax.experimental import pallas as pl


def kernel(x, normalfeature, pointfusefeature, W1, b1, g1, be1, W2, b2, g2, be2, W3, b3, g3, be3, W4, b4, g4, be4, Wf1, bf1, gf1, bef1, Wf2, bf2, gf2, bef2, Wf3, bf3):
    raise NotImplementedError("write your pallas kernel here")


# speedup vs baseline: 12.6351x; 12.6351x over previous
import functools

import jax
import jax.numpy as jnp
from jax import lax
from jax.experimental import pallas as pl
from jax.experimental.pallas import tpu as pltpu
from jax.experimental.pallas import tpu_sc as plsc

EPS = 1e-5
B = 8
N = 2048
K = 8
RT = 256
NT = 512
NC = 2
NS = 16
NW = NC * NS


def _lrelu(t):
    return jnp.maximum(t, 0.2 * t)


def _stage1_body(x_ref, nf_ref, pf_ref, w1_ref, y_ref, ssum_ref, ssq_ref):
    b = pl.program_id(0)
    nt = pl.program_id(1)
    xcat = jnp.concatenate([x_ref[0] + nf_ref[0], pf_ref[0]], axis=0)
    y = lax.dot_general(xcat, w1_ref[...], (((0,), (0,)), ((), ())),
                        preferred_element_type=jnp.float32)
    y_ref[0] = y
    s = jnp.sum(y, axis=0, keepdims=True)
    s2 = jnp.sum(y * y, axis=0, keepdims=True)
    first = jnp.logical_and(b == 0, nt == 0)

    @pl.when(first)
    def _():
        ssum_ref[...] = s
        ssq_ref[...] = s2

    @pl.when(jnp.logical_not(first))
    def _():
        ssum_ref[...] = ssum_ref[...] + s
        ssq_ref[...] = ssq_ref[...] + s2


def _stage1(x, nf, pf, w1):
    return pl.pallas_call(
        _stage1_body,
        grid=(B, N // NT),
        in_specs=[
            pl.BlockSpec((1, 128, NT), lambda b, nt: (b, 0, nt)),
            pl.BlockSpec((1, 128, NT), lambda b, nt: (b, 0, nt)),
            pl.BlockSpec((1, 128, NT), lambda b, nt: (b, 0, nt)),
            pl.BlockSpec((256, 128), lambda b, nt: (0, 0)),
        ],
        out_specs=[
            pl.BlockSpec((1, NT, 128), lambda b, nt: (b, nt, 0)),
            pl.BlockSpec((1, 128), lambda b, nt: (0, 0)),
            pl.BlockSpec((1, 128), lambda b, nt: (0, 0)),
        ],
        out_shape=[
            jax.ShapeDtypeStruct((B, N, 128), jnp.float32),
            jax.ShapeDtypeStruct((1, 128), jnp.float32),
            jax.ShapeDtypeStruct((1, 128), jnp.float32),
        ],
    )(x, nf, pf, w1)


def _knn_body(c_in, ytfull_ref, yrows_ref, sct_ref, offt_ref, sc_ref, off_ref,
              idx_ref, tab_ref):
    b = pl.program_id(0)
    ft_full = _lrelu(ytfull_ref[0] * sct_ref[...] + offt_ref[...])
    f_rows = _lrelu(yrows_ref[0] * sc_ref[...] + off_ref[...])
    g = lax.dot_general(f_rows, ft_full, (((1,), (0,)), ((), ())),
                        preferred_element_type=jnp.float32)
    nc = jnp.sum(ft_full * ft_full, axis=0, keepdims=True)
    nr = jnp.sum(f_rows * f_rows, axis=1, keepdims=True)
    p = 2.0 * g - nr - nc

    colidx = lax.broadcasted_iota(jnp.int32, (RT, N), 1)
    slot = lax.broadcasted_iota(jnp.int32, (RT, K), 1)
    idx = jnp.zeros((RT, K), jnp.int32)
    neg = jnp.float32(-jnp.inf)
    for t in range(K):
        m = jnp.max(p, axis=1, keepdims=True)
        cand = jnp.where(p == m, colidx, N)
        j = jnp.min(cand, axis=1, keepdims=True)
        idx = jnp.where(slot == t, jnp.broadcast_to(j, (RT, K)), idx)
        p = jnp.where(colidx == j, neg, p)
    idx_ref[0] = idx + b * N

    if c_in == 128:
        tab_ref[0] = f_rows
    else:
        tab_ref[0] = jnp.concatenate(
            [f_rows, jnp.zeros((RT, 128 - c_in), jnp.float32)], axis=1)


def _knn_stage(y, sc, off, c_in):
    yt = jnp.transpose(y, (0, 2, 1))
    return pl.pallas_call(
        functools.partial(_knn_body, c_in),
        grid=(B, N // RT),
        in_specs=[
            pl.BlockSpec((1, c_in, N), lambda b, rt: (b, 0, 0)),
            pl.BlockSpec((1, RT, c_in), lambda b, rt: (b, rt, 0)),
            pl.BlockSpec((c_in, 1), lambda b, rt: (0, 0)),
            pl.BlockSpec((c_in, 1), lambda b, rt: (0, 0)),
            pl.BlockSpec((1, c_in), lambda b, rt: (0, 0)),
            pl.BlockSpec((1, c_in), lambda b, rt: (0, 0)),
        ],
        out_specs=[
            pl.BlockSpec((1, RT, K), lambda b, rt: (b, rt, 0)),
            pl.BlockSpec((1, RT, 128), lambda b, rt: (b, rt, 0)),
        ],
        out_shape=[
            jax.ShapeDtypeStruct((B, N, K), jnp.int32),
            jax.ShapeDtypeStruct((B, N, 128), jnp.float32),
        ],
    )(yt, y, sc.reshape(-1, 1), off.reshape(-1, 1), sc, off)


def _sc_gather(table, idx):
    e = idx.shape[0]
    per_w = e // NW
    ch = 128
    nchunks = per_w // ch
    mesh = plsc.VectorSubcoreMesh(core_axis_name="c", subcore_axis_name="s")

    @functools.partial(
        pl.kernel,
        mesh=mesh,
        out_type=jax.ShapeDtypeStruct((e, 128), jnp.float32),
        scratch_types=[
            pltpu.VMEM((per_w,), jnp.int32),
            pltpu.VMEM((ch, 128), jnp.float32),
            pltpu.SemaphoreType.DMA,
        ],
    )
    def gather_kernel(table_hbm, idx_hbm, out_hbm, idx_v, rows_v, sem):
        wid = lax.axis_index("s") * NC + lax.axis_index("c")
        base = pl.multiple_of(wid * per_w, 128)
        pltpu.sync_copy(idx_hbm.at[pl.ds(base, per_w)], idx_v)

        def body(ci, carry):
            off = pl.multiple_of(ci * ch, 128)
            pltpu.async_copy(table_hbm.at[idx_v.at[pl.ds(off, ch)]],
                             rows_v, sem).wait()
            pltpu.sync_copy(rows_v, out_hbm.at[pl.ds(base + off, ch)])
            return carry

        lax.fori_loop(0, nchunks, body, 0)

    return gather_kernel(table, idx)


def _edge_body(c_e, g_ref, f_ref, w_ref, mt_ref, st_ref, st2_ref):
    b = pl.program_id(0)
    nt = pl.program_id(1)
    fi = f_ref[0][:, :c_e]
    mt = None
    s1 = None
    s2 = None
    for j in range(K):
        fj = g_ref[j, 0][:, :c_e]
        e = jnp.concatenate([fj - fi, fi], axis=1)
        t = lax.dot_general(e, w_ref[...], (((1,), (0,)), ((), ())),
                            preferred_element_type=jnp.float32)
        if mt is None:
            mt, s1, s2 = t, t, t * t
        else:
            mt = jnp.maximum(mt, t)
            s1 = s1 + t
            s2 = s2 + t * t
    mt_ref[0] = mt
    st = jnp.sum(s1, axis=0, keepdims=True)
    st2 = jnp.sum(s2, axis=0, keepdims=True)
    first = jnp.logical_and(b == 0, nt == 0)

    @pl.when(first)
    def _():
        st_ref[...] = st
        st2_ref[...] = st2

    @pl.when(jnp.logical_not(first))
    def _():
        st_ref[...] = st_ref[...] + st
        st2_ref[...] = st2_ref[...] + st2


def _edge_stage(gathered, f, wt, c_e, c_out):
    return pl.pallas_call(
        functools.partial(_edge_body, c_e),
        grid=(B, N // RT),
        in_specs=[
            pl.BlockSpec((K, 1, RT, 128), lambda b, nt: (0, b, nt, 0)),
            pl.BlockSpec((1, RT, 128), lambda b, nt: (b, nt, 0)),
            pl.BlockSpec((2 * c_e, c_out), lambda b, nt: (0, 0)),
        ],
        out_specs=[
            pl.BlockSpec((1, RT, c_out), lambda b, nt: (b, nt, 0)),
            pl.BlockSpec((1, c_out), lambda b, nt: (0, 0)),
            pl.BlockSpec((1, c_out), lambda b, nt: (0, 0)),
        ],
        out_shape=[
            jax.ShapeDtypeStruct((B, N, c_out), jnp.float32),
            jax.ShapeDtypeStruct((1, c_out), jnp.float32),
            jax.ShapeDtypeStruct((1, c_out), jnp.float32),
        ],
    )(gathered, f, wt)


def _stage4_body(feat_ref, u2_ref, sc_ref, off_ref, w4_ref,
                 dmax_ref, ssum_ref, ssq_ref):
    b = pl.program_id(0)
    nt = pl.program_id(1)
    h = feat_ref[0] + _lrelu(u2_ref[0] * sc_ref[...] + off_ref[...])
    y = lax.dot_general(h, w4_ref[...], (((1,), (0,)), ((), ())),
                        preferred_element_type=jnp.float32)
    tmax = jnp.max(y, axis=0, keepdims=True)
    s = jnp.sum(y, axis=0, keepdims=True)
    s2 = jnp.sum(y * y, axis=0, keepdims=True)

    @pl.when(nt == 0)
    def _():
        dmax_ref[0] = tmax

    @pl.when(nt != 0)
    def _():
        dmax_ref[0] = jnp.maximum(dmax_ref[0], tmax)

    first = jnp.logical_and(b == 0, nt == 0)

    @pl.when(first)
    def _():
        ssum_ref[...] = s
        ssq_ref[...] = s2

    @pl.when(jnp.logical_not(first))
    def _():
        ssum_ref[...] = ssum_ref[...] + s
        ssq_ref[...] = ssq_ref[...] + s2


def _stage4(feat, u2, sc, off, w4):
    return pl.pallas_call(
        _stage4_body,
        grid=(B, N // NT),
        in_specs=[
            pl.BlockSpec((1, NT, 128), lambda b, nt: (b, nt, 0)),
            pl.BlockSpec((1, NT, 128), lambda b, nt: (b, nt, 0)),
            pl.BlockSpec((1, 128), lambda b, nt: (0, 0)),
            pl.BlockSpec((1, 128), lambda b, nt: (0, 0)),
            pl.BlockSpec((128, 256), lambda b, nt: (0, 0)),
        ],
        out_specs=[
            pl.BlockSpec((1, 1, 256), lambda b, nt: (b, 0, 0)),
            pl.BlockSpec((1, 256), lambda b, nt: (0, 0)),
            pl.BlockSpec((1, 256), lambda b, nt: (0, 0)),
        ],
        out_shape=[
            jax.ShapeDtypeStruct((B, 1, 256), jnp.float32),
            jax.ShapeDtypeStruct((1, 256), jnp.float32),
            jax.ShapeDtypeStruct((1, 256), jnp.float32),
        ],
    )(feat, u2, sc, off, w4)


def _head_body(d_ref, sc_ref, off_ref, wf1_ref, gf1_ref, bef1_ref,
               wf2_ref, gf2_ref, bef2_ref, wf3_ref, bf3_ref, out_ref):
    d = _lrelu(d_ref[...] * sc_ref[...] + off_ref[...])
    h1 = lax.dot_general(d, wf1_ref[...], (((1,), (0,)), ((), ())),
                         preferred_element_type=jnp.float32)
    m = jnp.mean(h1, axis=0, keepdims=True)
    v = jnp.mean(h1 * h1, axis=0, keepdims=True) - m * m
    h1 = _lrelu((h1 - m) * lax.rsqrt(v + EPS) * gf1_ref[...] + bef1_ref[...])
    h2 = lax.dot_general(h1, wf2_ref[...], (((1,), (0,)), ((), ())),
                         preferred_element_type=jnp.float32)
    m = jnp.mean(h2, axis=0, keepdims=True)
    v = jnp.mean(h2 * h2, axis=0, keepdims=True) - m * m
    h2 = _lrelu((h2 - m) * lax.rsqrt(v + EPS) * gf2_ref[...] + bef2_ref[...])
    out_ref[...] = lax.dot_general(h2, wf3_ref[...], (((1,), (0,)), ((), ())),
                                   preferred_element_type=jnp.float32) \
        + bf3_ref[...]


def _head(d, sc, off, wf1, gf1, bef1, wf2, gf2, bef2, wf3p, bf3p):
    return pl.pallas_call(
        _head_body,
        grid=(1,),
        in_specs=[
            pl.BlockSpec((B, 256), lambda i: (0, 0)),
            pl.BlockSpec((1, 256), lambda i: (0, 0)),
            pl.BlockSpec((1, 256), lambda i: (0, 0)),
            pl.BlockSpec((256, 128), lambda i: (0, 0)),
            pl.BlockSpec((1, 128), lambda i: (0, 0)),
            pl.BlockSpec((1, 128), lambda i: (0, 0)),
            pl.BlockSpec((128, 64), lambda i: (0, 0)),
            pl.BlockSpec((1, 64), lambda i: (0, 0)),
            pl.BlockSpec((1, 64), lambda i: (0, 0)),
            pl.BlockSpec((64, 128), lambda i: (0, 0)),
            pl.BlockSpec((1, 128), lambda i: (0, 0)),
        ],
        out_specs=pl.BlockSpec((B, 128), lambda i: (0, 0)),
        out_shape=jax.ShapeDtypeStruct((B, 128), jnp.float32),
    )(d, sc, off, wf1, gf1, bef1, wf2, gf2, bef2, wf3p, bf3p)


def _scale_offset(ssum, ssq, cnt, g, be):
    mean = ssum[0] / cnt
    var = ssq[0] / cnt - mean * mean
    sc = g / jnp.sqrt(var + EPS)
    off = be - mean * sc
    return sc.reshape(1, -1), off.reshape(1, -1)


def _slot_major(idx):
    return jnp.transpose(idx, (2, 0, 1)).reshape(-1)


def kernel(x, normalfeature, pointfusefeature, W1, b1, g1, be1, W2, b2, g2, be2,
           W3, b3, g3, be3, W4, b4, g4, be4, Wf1, bf1, gf1, bef1, Wf2, bf2, gf2,
           bef2, Wf3, bf3):
    w1 = jnp.transpose(W1)
    w2 = jnp.transpose(W2)
    w3 = jnp.transpose(W3)
    w4 = jnp.transpose(W4)
    wf1 = jnp.transpose(Wf1)
    wf2 = jnp.transpose(Wf2)
    wf3p = jnp.zeros((64, 128), jnp.float32).at[:, :3].set(jnp.transpose(Wf3))
    bf3p = jnp.zeros((1, 128), jnp.float32).at[0, :3].set(bf3)

    y1, s1, q1 = _stage1(x, normalfeature, pointfusefeature, w1)
    sc1, off1 = _scale_offset(s1, q1, B * N, g1, be1)

    idx1, feat = _knn_stage(y1, sc1, off1, 128)
    eg1 = _sc_gather(feat.reshape(B * N, 128), _slot_major(idx1))
    mt1, s2_, q2 = _edge_stage(eg1.reshape(K, B, N, 128), feat, w2, 128, 64)
    sc2, off2 = _scale_offset(s2_, q2, B * N * K, g2, be2)

    idx2, f1pad = _knn_stage(mt1, sc2, off2, 64)
    eg2 = _sc_gather(f1pad.reshape(B * N, 128), _slot_major(idx2))
    mt2, s3, q3 = _edge_stage(eg2.reshape(K, B, N, 128), f1pad, w3, 64, 128)
    sc3, off3 = _scale_offset(s3, q3, B * N * K, g3, be3)

    dmax, s4, q4 = _stage4(feat, mt2, sc3, off3, w4)
    sc4, off4 = _scale_offset(s4, q4, B * N, g4, be4)

    out = _head(dmax.reshape(B, 256), sc4, off4,
                wf1, gf1.reshape(1, -1), bef1.reshape(1, -1),
                wf2, gf2.reshape(1, -1), bef2.reshape(1, -1),
                wf3p, bf3p)
    return out[:, :3]


# speedup vs baseline: 12.7257x; 1.0072x over previous
import functools

import jax
import jax.numpy as jnp
from jax import lax
from jax.experimental import pallas as pl
from jax.experimental.pallas import tpu as pltpu
from jax.experimental.pallas import tpu_sc as plsc

EPS = 1e-5
B = 8
N = 2048
K = 8
RT = 256
NT = 512
NC = 2
NS = 16
NW = NC * NS


def _lrelu(t):
    return jnp.maximum(t, 0.2 * t)


def _stage1_body(x_ref, nf_ref, pf_ref, w1_ref, y_ref, ssum_ref, ssq_ref):
    b = pl.program_id(0)
    nt = pl.program_id(1)
    xcat = jnp.concatenate([x_ref[0] + nf_ref[0], pf_ref[0]], axis=0)
    y = lax.dot_general(w1_ref[...], xcat, (((1,), (0,)), ((), ())),
                        preferred_element_type=jnp.float32)
    y_ref[0] = y
    s = jnp.sum(y, axis=1, keepdims=True)
    s2 = jnp.sum(y * y, axis=1, keepdims=True)
    first = jnp.logical_and(b == 0, nt == 0)

    @pl.when(first)
    def _():
        ssum_ref[...] = s
        ssq_ref[...] = s2

    @pl.when(jnp.logical_not(first))
    def _():
        ssum_ref[...] = ssum_ref[...] + s
        ssq_ref[...] = ssq_ref[...] + s2


def _stage1(x, nf, pf, w1):
    return pl.pallas_call(
        _stage1_body,
        grid=(B, N // NT),
        in_specs=[
            pl.BlockSpec((1, 128, NT), lambda b, nt: (b, 0, nt)),
            pl.BlockSpec((1, 128, NT), lambda b, nt: (b, 0, nt)),
            pl.BlockSpec((1, 128, NT), lambda b, nt: (b, 0, nt)),
            pl.BlockSpec((128, 256), lambda b, nt: (0, 0)),
        ],
        out_specs=[
            pl.BlockSpec((1, 128, NT), lambda b, nt: (b, 0, nt)),
            pl.BlockSpec((128, 1), lambda b, nt: (0, 0)),
            pl.BlockSpec((128, 1), lambda b, nt: (0, 0)),
        ],
        out_shape=[
            jax.ShapeDtypeStruct((B, 128, N), jnp.float32),
            jax.ShapeDtypeStruct((128, 1), jnp.float32),
            jax.ShapeDtypeStruct((128, 1), jnp.float32),
        ],
    )(x, nf, pf, w1)


def _knn_body(c_in, ytfull_ref, ytrows_ref, sct_ref, offt_ref,
              idx_ref, tab_ref):
    b = pl.program_id(0)
    ft_full = _lrelu(ytfull_ref[0] * sct_ref[...] + offt_ref[...])
    ft_rows = _lrelu(ytrows_ref[0] * sct_ref[...] + offt_ref[...])
    f_rows = jnp.transpose(ft_rows)
    g = lax.dot_general(f_rows, ft_full, (((1,), (0,)), ((), ())),
                        preferred_element_type=jnp.float32)
    nc = jnp.sum(ft_full * ft_full, axis=0, keepdims=True)
    nr = jnp.sum(f_rows * f_rows, axis=1, keepdims=True)
    p = 2.0 * g - nr - nc

    colidx = lax.broadcasted_iota(jnp.int32, (RT, N), 1)
    slot = lax.broadcasted_iota(jnp.int32, (RT, K), 1)
    idx = jnp.zeros((RT, K), jnp.int32)
    neg = jnp.float32(-jnp.inf)
    for t in range(K):
        m = jnp.max(p, axis=1, keepdims=True)
        cand = jnp.where(p == m, colidx, N)
        j = jnp.min(cand, axis=1, keepdims=True)
        idx = jnp.where(slot == t, jnp.broadcast_to(j, (RT, K)), idx)
        p = jnp.where(colidx == j, neg, p)
    idx_ref[0] = idx + b * N

    if c_in == 128:
        tab_ref[0] = f_rows
    else:
        tab_ref[0] = jnp.concatenate(
            [f_rows, jnp.zeros((RT, 128 - c_in), jnp.float32)], axis=1)


def _knn_stage(yt, sc, off, c_in):
    return pl.pallas_call(
        functools.partial(_knn_body, c_in),
        grid=(B, N // RT),
        in_specs=[
            pl.BlockSpec((1, c_in, N), lambda b, rt: (b, 0, 0)),
            pl.BlockSpec((1, c_in, RT), lambda b, rt: (b, 0, rt)),
            pl.BlockSpec((c_in, 1), lambda b, rt: (0, 0)),
            pl.BlockSpec((c_in, 1), lambda b, rt: (0, 0)),
        ],
        out_specs=[
            pl.BlockSpec((1, RT, K), lambda b, rt: (b, rt, 0)),
            pl.BlockSpec((1, RT, 128), lambda b, rt: (b, rt, 0)),
        ],
        out_shape=[
            jax.ShapeDtypeStruct((B, N, K), jnp.int32),
            jax.ShapeDtypeStruct((B, N, 128), jnp.float32),
        ],
    )(yt, yt, sc.reshape(-1, 1), off.reshape(-1, 1))


def _sc_gather(table, idx):
    e = idx.shape[0]
    per_w = e // NW
    ch = 128
    nchunks = per_w // ch
    mesh = plsc.VectorSubcoreMesh(core_axis_name="c", subcore_axis_name="s")

    @functools.partial(
        pl.kernel,
        mesh=mesh,
        out_type=jax.ShapeDtypeStruct((e, 128), jnp.float32),
        scratch_types=[
            pltpu.VMEM((per_w,), jnp.int32),
            pltpu.VMEM((ch, 128), jnp.float32),
            pltpu.SemaphoreType.DMA,
        ],
    )
    def gather_kernel(table_hbm, idx_hbm, out_hbm, idx_v, rows_v, sem):
        wid = lax.axis_index("s") * NC + lax.axis_index("c")
        base = pl.multiple_of(wid * per_w, 128)
        pltpu.sync_copy(idx_hbm.at[pl.ds(base, per_w)], idx_v)

        def body(ci, carry):
            off = pl.multiple_of(ci * ch, 128)
            pltpu.async_copy(table_hbm.at[idx_v.at[pl.ds(off, ch)]],
                             rows_v, sem).wait()
            pltpu.sync_copy(rows_v, out_hbm.at[pl.ds(base + off, ch)])
            return carry

        lax.fori_loop(0, nchunks, body, 0)

    return gather_kernel(table, idx)


def _edge_body(c_e, g_ref, f_ref, w_ref, mt_ref, st_ref, st2_ref):
    b = pl.program_id(0)
    nt = pl.program_id(1)
    fi = f_ref[0][:, :c_e]
    mt = None
    s1 = None
    s2 = None
    for j in range(K):
        fj = g_ref[j, 0][:, :c_e]
        e = jnp.concatenate([fj - fi, fi], axis=1)
        t = lax.dot_general(e, w_ref[...], (((1,), (0,)), ((), ())),
                            preferred_element_type=jnp.float32)
        if mt is None:
            mt, s1, s2 = t, t, t * t
        else:
            mt = jnp.maximum(mt, t)
            s1 = s1 + t
            s2 = s2 + t * t
    mt_ref[0] = mt
    st = jnp.sum(s1, axis=0, keepdims=True)
    st2 = jnp.sum(s2, axis=0, keepdims=True)
    first = jnp.logical_and(b == 0, nt == 0)

    @pl.when(first)
    def _():
        st_ref[...] = st
        st2_ref[...] = st2

    @pl.when(jnp.logical_not(first))
    def _():
        st_ref[...] = st_ref[...] + st
        st2_ref[...] = st2_ref[...] + st2


def _edge_stage(gathered, f, wt, c_e, c_out):
    return pl.pallas_call(
        functools.partial(_edge_body, c_e),
        grid=(B, N // RT),
        in_specs=[
            pl.BlockSpec((K, 1, RT, 128), lambda b, nt: (0, b, nt, 0)),
            pl.BlockSpec((1, RT, 128), lambda b, nt: (b, nt, 0)),
            pl.BlockSpec((2 * c_e, c_out), lambda b, nt: (0, 0)),
        ],
        out_specs=[
            pl.BlockSpec((1, RT, c_out), lambda b, nt: (b, nt, 0)),
            pl.BlockSpec((1, c_out), lambda b, nt: (0, 0)),
            pl.BlockSpec((1, c_out), lambda b, nt: (0, 0)),
        ],
        out_shape=[
            jax.ShapeDtypeStruct((B, N, c_out), jnp.float32),
            jax.ShapeDtypeStruct((1, c_out), jnp.float32),
            jax.ShapeDtypeStruct((1, c_out), jnp.float32),
        ],
    )(gathered, f, wt)


def _stage4_body(feat_ref, u2_ref, sc_ref, off_ref, w4_ref,
                 dmax_ref, ssum_ref, ssq_ref):
    b = pl.program_id(0)
    nt = pl.program_id(1)
    h = feat_ref[0] + _lrelu(u2_ref[0] * sc_ref[...] + off_ref[...])
    y = lax.dot_general(h, w4_ref[...], (((1,), (0,)), ((), ())),
                        preferred_element_type=jnp.float32)
    tmax = jnp.max(y, axis=0, keepdims=True)
    s = jnp.sum(y, axis=0, keepdims=True)
    s2 = jnp.sum(y * y, axis=0, keepdims=True)

    @pl.when(nt == 0)
    def _():
        dmax_ref[0] = tmax

    @pl.when(nt != 0)
    def _():
        dmax_ref[0] = jnp.maximum(dmax_ref[0], tmax)

    first = jnp.logical_and(b == 0, nt == 0)

    @pl.when(first)
    def _():
        ssum_ref[...] = s
        ssq_ref[...] = s2

    @pl.when(jnp.logical_not(first))
    def _():
        ssum_ref[...] = ssum_ref[...] + s
        ssq_ref[...] = ssq_ref[...] + s2


def _stage4(feat, u2, sc, off, w4):
    return pl.pallas_call(
        _stage4_body,
        grid=(B, N // NT),
        in_specs=[
            pl.BlockSpec((1, NT, 128), lambda b, nt: (b, nt, 0)),
            pl.BlockSpec((1, NT, 128), lambda b, nt: (b, nt, 0)),
            pl.BlockSpec((1, 128), lambda b, nt: (0, 0)),
            pl.BlockSpec((1, 128), lambda b, nt: (0, 0)),
            pl.BlockSpec((128, 256), lambda b, nt: (0, 0)),
        ],
        out_specs=[
            pl.BlockSpec((1, 1, 256), lambda b, nt: (b, 0, 0)),
            pl.BlockSpec((1, 256), lambda b, nt: (0, 0)),
            pl.BlockSpec((1, 256), lambda b, nt: (0, 0)),
        ],
        out_shape=[
            jax.ShapeDtypeStruct((B, 1, 256), jnp.float32),
            jax.ShapeDtypeStruct((1, 256), jnp.float32),
            jax.ShapeDtypeStruct((1, 256), jnp.float32),
        ],
    )(feat, u2, sc, off, w4)


def _head_body(d_ref, sc_ref, off_ref, wf1_ref, gf1_ref, bef1_ref,
               wf2_ref, gf2_ref, bef2_ref, wf3_ref, bf3_ref, out_ref):
    d = _lrelu(d_ref[...] * sc_ref[...] + off_ref[...])
    h1 = lax.dot_general(d, wf1_ref[...], (((1,), (0,)), ((), ())),
                         preferred_element_type=jnp.float32)
    m = jnp.mean(h1, axis=0, keepdims=True)
    v = jnp.mean(h1 * h1, axis=0, keepdims=True) - m * m
    h1 = _lrelu((h1 - m) * lax.rsqrt(v + EPS) * gf1_ref[...] + bef1_ref[...])
    h2 = lax.dot_general(h1, wf2_ref[...], (((1,), (0,)), ((), ())),
                         preferred_element_type=jnp.float32)
    m = jnp.mean(h2, axis=0, keepdims=True)
    v = jnp.mean(h2 * h2, axis=0, keepdims=True) - m * m
    h2 = _lrelu((h2 - m) * lax.rsqrt(v + EPS) * gf2_ref[...] + bef2_ref[...])
    out_ref[...] = lax.dot_general(h2, wf3_ref[...], (((1,), (0,)), ((), ())),
                                   preferred_element_type=jnp.float32) \
        + bf3_ref[...]


def _head(d, sc, off, wf1, gf1, bef1, wf2, gf2, bef2, wf3p, bf3p):
    return pl.pallas_call(
        _head_body,
        grid=(1,),
        in_specs=[
            pl.BlockSpec((B, 256), lambda i: (0, 0)),
            pl.BlockSpec((1, 256), lambda i: (0, 0)),
            pl.BlockSpec((1, 256), lambda i: (0, 0)),
            pl.BlockSpec((256, 128), lambda i: (0, 0)),
            pl.BlockSpec((1, 128), lambda i: (0, 0)),
            pl.BlockSpec((1, 128), lambda i: (0, 0)),
            pl.BlockSpec((128, 64), lambda i: (0, 0)),
            pl.BlockSpec((1, 64), lambda i: (0, 0)),
            pl.BlockSpec((1, 64), lambda i: (0, 0)),
            pl.BlockSpec((64, 128), lambda i: (0, 0)),
            pl.BlockSpec((1, 128), lambda i: (0, 0)),
        ],
        out_specs=pl.BlockSpec((B, 128), lambda i: (0, 0)),
        out_shape=jax.ShapeDtypeStruct((B, 128), jnp.float32),
    )(d, sc, off, wf1, gf1, bef1, wf2, gf2, bef2, wf3p, bf3p)


def _scale_offset(ssum, ssq, cnt, g, be):
    mean = ssum[0] / cnt
    var = ssq[0] / cnt - mean * mean
    sc = g / jnp.sqrt(var + EPS)
    off = be - mean * sc
    return sc.reshape(1, -1), off.reshape(1, -1)


def _slot_major(idx):
    return jnp.transpose(idx, (2, 0, 1)).reshape(-1)


def kernel(x, normalfeature, pointfusefeature, W1, b1, g1, be1, W2, b2, g2, be2,
           W3, b3, g3, be3, W4, b4, g4, be4, Wf1, bf1, gf1, bef1, Wf2, bf2, gf2,
           bef2, Wf3, bf3):
    w2 = jnp.transpose(W2)
    w3 = jnp.transpose(W3)
    w4 = jnp.transpose(W4)
    wf1 = jnp.transpose(Wf1)
    wf2 = jnp.transpose(Wf2)
    wf3p = jnp.zeros((64, 128), jnp.float32).at[:, :3].set(jnp.transpose(Wf3))
    bf3p = jnp.zeros((1, 128), jnp.float32).at[0, :3].set(bf3)

    y1, s1, q1 = _stage1(x, normalfeature, pointfusefeature, W1)
    sc1, off1 = _scale_offset(s1.reshape(1, -1), q1.reshape(1, -1), B * N,
                              g1, be1)

    idx1, feat = _knn_stage(y1, sc1, off1, 128)
    eg1 = _sc_gather(feat.reshape(B * N, 128), _slot_major(idx1))
    mt1, s2_, q2 = _edge_stage(eg1.reshape(K, B, N, 128), feat, w2, 128, 64)
    sc2, off2 = _scale_offset(s2_, q2, B * N * K, g2, be2)

    idx2, f1pad = _knn_stage(jnp.transpose(mt1, (0, 2, 1)), sc2, off2, 64)
    eg2 = _sc_gather(f1pad.reshape(B * N, 128), _slot_major(idx2))
    mt2, s3, q3 = _edge_stage(eg2.reshape(K, B, N, 128), f1pad, w3, 64, 128)
    sc3, off3 = _scale_offset(s3, q3, B * N * K, g3, be3)

    dmax, s4, q4 = _stage4(feat, mt2, sc3, off3, w4)
    sc4, off4 = _scale_offset(s4, q4, B * N, g4, be4)

    out = _head(dmax.reshape(B, 256), sc4, off4,
                wf1, gf1.reshape(1, -1), bef1.reshape(1, -1),
                wf2, gf2.reshape(1, -1), bef2.reshape(1, -1),
                wf3p, bf3p)
    return out[:, :3]
